# Initial kernel scaffold; baseline (speedup 1.0000x reference)
import jax
import jax.numpy as jnp
from jax.experimental import pallas as pl


def kernel(x, edge_index, batch, W_in, b_in, W_1, b_1, Wl_val, bl_val, Wr_val, W_pol, b_pol):
    raise NotImplementedError("write your pallas kernel here")


# speedup vs baseline: 8.1789x; 8.1789x over previous
import functools

import jax
import jax.numpy as jnp
from jax import lax
from jax.experimental import pallas as pl
from jax.experimental.pallas import tpu as pltpu
from jax.experimental.pallas import tpu_sc as plsc

N = 10000
E = 320000
F = 128
G = 16
MAXD = 10
KST = MAXD + 2
KW = 208
NPAD = 10240
NC, NS = 2, 16
NW = NC * NS
CH = 128
KCH = (E + NW * CH - 1) // (NW * CH)
EP = NW * CH * KCH
RPS = NPAD // NS
BN = 1024
GRID = NPAD // BN


def _mesh():
    return plsc.VectorSubcoreMesh(core_axis_name="c", subcore_axis_name="s",
                                  num_cores=NC, num_subcores=NS)


DW = 128


def _sc_deg_body(dst_hbm, ones_hbm, zero_hbm, out_hbm, didx, ones_v, acc_sh):
    c = lax.axis_index("c")
    s = lax.axis_index("s")
    wid = c * NS + s
    pltpu.sync_copy(dst_hbm.at[wid], didx)
    pltpu.sync_copy(ones_hbm, ones_v)
    pltpu.sync_copy(zero_hbm, acc_sh.at[pl.ds(s * RPS, RPS)])
    plsc.subcore_barrier()

    def chunk(j, carry):
        pltpu.sync_copy(ones_v, acc_sh.at[didx.at[j]], add=True)
        return carry

    lax.fori_loop(0, KCH, chunk, 0)
    plsc.subcore_barrier()
    pltpu.sync_copy(acc_sh.at[pl.ds(s * RPS, RPS)],
                    out_hbm.at[c].at[pl.ds(s * RPS, RPS)])


def _sc_deg(dstp):
    ones = jnp.ones((CH, DW), jnp.float32)
    zero = jnp.zeros((RPS, DW), jnp.float32)
    f = pl.kernel(
        _sc_deg_body,
        out_type=jax.ShapeDtypeStruct((NC, NPAD, DW), jnp.float32),
        mesh=_mesh(),
        scratch_types=[
            pltpu.VMEM((KCH, CH), jnp.int32),
            pltpu.VMEM((CH, DW), jnp.float32),
            pltpu.VMEM_SHARED((NPAD, DW), jnp.float32),
        ],
    )
    return f(dstp, ones, zero)


def _sc_pass_body(src_hbm, dst_hbm, feat_hbm, zero_hbm, out_hbm,
                  sidx, didx, rows, acc_sh, sem):
    c = lax.axis_index("c")
    s = lax.axis_index("s")
    wid = c * NS + s
    pltpu.sync_copy(src_hbm.at[wid], sidx)
    pltpu.sync_copy(dst_hbm.at[wid], didx)
    pltpu.sync_copy(zero_hbm, acc_sh.at[pl.ds(s * RPS, RPS)])
    plsc.subcore_barrier()

    def chunk(j, carry):
        pltpu.async_copy(feat_hbm.at[sidx.at[j]], rows, sem).wait()
        pltpu.sync_copy(rows, acc_sh.at[didx.at[j]], add=True)
        return carry

    lax.fori_loop(0, KCH, chunk, 0)
    plsc.subcore_barrier()
    pltpu.sync_copy(acc_sh.at[pl.ds(s * RPS, RPS)],
                    out_hbm.at[c].at[pl.ds(s * RPS, RPS)])


def _sc_pass(srcp, dstp, feat, zero640):
    f = pl.kernel(
        _sc_pass_body,
        out_type=jax.ShapeDtypeStruct((NC, NPAD, F), jnp.float32),
        mesh=_mesh(),
        scratch_types=[
            pltpu.VMEM((KCH, CH), jnp.int32),
            pltpu.VMEM((KCH, CH), jnp.int32),
            pltpu.VMEM((CH, F), jnp.float32),
            pltpu.VMEM_SHARED((NPAD, F), jnp.float32),
            pltpu.SemaphoreType.DMA,
        ],
    )
    return f(srcp, dstp, feat, zero640)


def _tcA_body(d0, d1, x, bt, xs1, dinv, key):
    deg = d0[:, :1] + d1[:, :1]
    din = lax.rsqrt(deg + 1.0)
    dinv[...] = din
    xs1[...] = x[...] * din
    dc = jnp.minimum(deg.astype(jnp.int32), MAXD)
    key[...] = bt[...] * KST + dc


def _tcA(degp, xp, btp):
    row1 = pl.BlockSpec((BN, 1), lambda i: (i, 0))
    rowD = pl.BlockSpec((BN, DW), lambda i: (i, 0))
    rowF = pl.BlockSpec((BN, F), lambda i: (i, 0))
    return pl.pallas_call(
        _tcA_body,
        grid=(GRID,),
        in_specs=[rowD, rowD, rowF, row1],
        out_specs=[rowF, row1, row1],
        out_shape=[jax.ShapeDtypeStruct((NPAD, F), jnp.float32),
                   jax.ShapeDtypeStruct((NPAD, 1), jnp.float32),
                   jax.ShapeDtypeStruct((NPAD, 1), jnp.int32)],
    )(degp[0], degp[1], xp, btp)


def _tcBC_body(a0, a1, xs, dinv, W, brow, h, xsn):
    p = dinv[...] * (a0[...] + a1[...] + xs[...])
    hv = jnp.dot(p, W[...], preferred_element_type=jnp.float32) + brow[0:1, :]
    h[...] = hv
    xsn[...] = dinv[...] * hv


def _tcBC(acc, xs, dinv, W, b):
    brow = jnp.zeros((8, F), jnp.float32).at[0].set(b)
    row1 = pl.BlockSpec((BN, 1), lambda i: (i, 0))
    rowF = pl.BlockSpec((BN, F), lambda i: (i, 0))
    full = lambda shape: pl.BlockSpec(shape, lambda i: tuple(0 for _ in shape))
    return pl.pallas_call(
        _tcBC_body,
        grid=(GRID,),
        in_specs=[rowF, rowF, rowF, row1, full((F, F)), full((8, F))],
        out_specs=[rowF, rowF],
        out_shape=[jax.ShapeDtypeStruct((NPAD, F), jnp.float32),
                   jax.ShapeDtypeStruct((NPAD, F), jnp.float32)],
    )(acc[0], acc[1], xs, dinv, W, brow)


def _tcD1_body(a0, a1, xs3, ah0, ah1, h2, dinv, bt, ky, P, T, S, cnt):
    i = pl.program_id(0)

    @pl.when(i == 0)
    def _():
        P[...] = jnp.zeros_like(P)
        T[...] = jnp.zeros_like(T)
        S[...] = jnp.zeros_like(S)
        cnt[...] = jnp.zeros_like(cnt)

    polnode = dinv[...] * (a0[...] + a1[...] + xs3[...])
    acch = ah0[...] + ah1[...]
    ob = (bt[...] == lax.broadcasted_iota(jnp.int32, (BN, G), 1)
          ).astype(jnp.float32)
    ok = (ky[...] == lax.broadcasted_iota(jnp.int32, (BN, KW), 1)
          ).astype(jnp.float32)
    dn = (((0,), (0,)), ((), ()))
    P[...] += lax.dot_general(ob, polnode, dn,
                              preferred_element_type=jnp.float32)
    T[...] += lax.dot_general(ok, h2[...], dn,
                              preferred_element_type=jnp.float32)
    S[...] += lax.dot_general(ok, acch, dn,
                              preferred_element_type=jnp.float32)
    cnt[...] += lax.dot_general(ok, jnp.ones((BN, 1), jnp.float32), dn,
                                preferred_element_type=jnp.float32)


def _tcD1(acc3, xs3, acch, h2, dinv, btp, key):
    row1 = pl.BlockSpec((BN, 1), lambda i: (i, 0))
    rowF = pl.BlockSpec((BN, F), lambda i: (i, 0))
    full = lambda shape: pl.BlockSpec(shape, lambda i: tuple(0 for _ in shape))
    return pl.pallas_call(
        _tcD1_body,
        grid=(GRID,),
        in_specs=[rowF, rowF, rowF, rowF, rowF, rowF, row1, row1, row1],
        out_specs=[full((G, F)), full((KW, F)), full((KW, F)), full((KW, 1))],
        out_shape=[jax.ShapeDtypeStruct((G, F), jnp.float32),
                   jax.ShapeDtypeStruct((KW, F), jnp.float32),
                   jax.ShapeDtypeStruct((KW, F), jnp.float32),
                   jax.ShapeDtypeStruct((KW, 1), jnp.float32)],
    )(acc3[0], acc3[1], xs3, acch[0], acch[1], h2, dinv, btp, key)


def _tcD2_body(P, T, S, c, wl, wr, bl, Gm, Wp, bp, pol, val):
    rv = jnp.sum(S[...] * wl[...] + T[...] * wr[...], axis=1, keepdims=True)
    rv = rv + c[...] * bl[...]
    val[...] = jnp.dot(Gm[...], rv, preferred_element_type=jnp.float32)
    counts = jnp.dot(Gm[...], c[...], preferred_element_type=jnp.float32)
    pv = jnp.dot(P[...], Wp[...], preferred_element_type=jnp.float32)
    pol[...] = pv / jnp.maximum(counts, 1.0) + bp[0:1, :]


def _tcD2(P, T, S, c, wlrep, wrrep, blrep, Gmat, W_pol, bp):
    full = lambda shape: pl.BlockSpec(shape, lambda: tuple(0 for _ in shape))
    return pl.pallas_call(
        _tcD2_body,
        in_specs=[full((G, F)), full((KW, F)), full((KW, F)), full((KW, 1)),
                  full((KW, F)), full((KW, F)), full((KW, 1)), full((G, KW)),
                  full((F, G)), full((8, G))],
        out_specs=[full((G, G)), full((G, 1))],
        out_shape=[jax.ShapeDtypeStruct((G, G), jnp.float32),
                   jax.ShapeDtypeStruct((G, 1), jnp.float32)],
    )(P, T, S, c, wlrep, wrrep, blrep, Gmat, W_pol, bp)


def kernel(x, edge_index, batch, W_in, b_in, W_1, b_1,
           Wl_val, bl_val, Wr_val, W_pol, b_pol):
    f32 = jnp.float32
    src = edge_index[0]
    dst = edge_index[1]
    pad_e = EP - E
    srcp = jnp.concatenate([src, jnp.full((pad_e,), N, jnp.int32)]
                           ).reshape(NW, KCH, CH)
    dstp = jnp.concatenate([dst, jnp.full((pad_e,), N, jnp.int32)]
                           ).reshape(NW, KCH, CH)
    xp = jnp.zeros((NPAD, F), f32).at[:N].set(x)
    btp = jnp.concatenate([batch.astype(jnp.int32),
                           jnp.full((NPAD - N,), G, jnp.int32)]
                          ).reshape(NPAD, 1)
    zero640 = jnp.zeros((RPS, F), f32)

    degp = _sc_deg(dstp)
    xs1, dinv, key = _tcA(degp, xp, btp)
    acc1 = _sc_pass(srcp, dstp, xs1, zero640)
    _h1, xs2 = _tcBC(acc1, xs1, dinv, W_in, b_in)
    acc2 = _sc_pass(srcp, dstp, xs2, zero640)
    h2, xs3 = _tcBC(acc2, xs2, dinv, W_1, b_1)
    acc3 = _sc_pass(srcp, dstp, xs3, zero640)
    acch = _sc_pass(srcp, dstp, h2, zero640)
    P, T, S, c = _tcD1(acc3, xs3, acch, h2, dinv, btp, key)

    wl = Wl_val[:, :, 0]
    wr = Wr_val[:, :, 0]
    bl = bl_val[:, 0]
    wl12 = jnp.zeros((KST, F), f32).at[:MAXD + 1].set(wl)
    wr12 = jnp.zeros((KST, F), f32).at[:MAXD + 1].set(wr)
    bl12 = jnp.zeros((KST,), f32).at[:MAXD + 1].set(bl)
    wlrep = jnp.zeros((KW, F), f32).at[:G * KST].set(jnp.tile(wl12, (G, 1)))
    wrrep = jnp.zeros((KW, F), f32).at[:G * KST].set(jnp.tile(wr12, (G, 1)))
    blrep = jnp.zeros((KW, 1), f32).at[:G * KST, 0].set(jnp.tile(bl12, G))
    col = jnp.arange(KW, dtype=jnp.int32)
    Gmat = (((col[None, :] // KST) == jnp.arange(G, dtype=jnp.int32)[:, None])
            & (col[None, :] < G * KST)).astype(f32)
    bp = jnp.zeros((8, G), f32).at[0].set(b_pol)

    pol, val = _tcD2(P, T, S, c, wlrep, wrrep, blrep, Gmat, W_pol, bp)
    return (pol, val)
